# Initial kernel scaffold; baseline (speedup 1.0000x reference)
import jax
import jax.numpy as jnp
from jax import lax
from jax.experimental import pallas as pl
from jax.experimental.pallas import tpu as pltpu

_NEG = jnp.float32(-1e30)


def _model_body(adj_ref, scal_ref, arr_ref, dep_ref, ihtm_ref, vcol_ref,
                vrow_ref, we1_ref, be1_ref, we2_ref, be2_ref, gw_ref,
                gas_ref, gad_ref, gb_ref, wsel_ref, out_ref):
    f32 = jnp.float32
    tsf = scal_ref[0, 0]
    cc = scal_ref[0, 1]

    arr = arr_ref[...]
    dep = dep_ref[...]
    ihtm = ihtm_ref[...]
    vcol = vcol_ref[...]
    vrow = vrow_ref[...]

    prog = (tsf - arr) / (dep - arr)
    x = prog * we1_ref[0:1, :] + ihtm * we1_ref[1:2, :] + be1_ref[...]
    x = jnp.dot(x, we2_ref[...], preferred_element_type=f32) + be2_ref[...]

    n = arr.shape[0]
    adjb = (adj_ref[...] > 0.0).astype(f32)
    ri = lax.broadcasted_iota(jnp.int32, (n, n), 0)
    ci = lax.broadcasted_iota(jnp.int32, (n, n), 1)
    diag = (ri == ci).astype(f32)
    cnt = adjb * (vcol * vrow) + diag * vrow
    valid_edge = cnt > 0.0

    ones_col = jnp.ones((n, 1), f32)

    h = x
    nlayers = gw_ref.shape[0]
    for l in range(nlayers):
        hp = jnp.dot(h, gw_ref[l], preferred_element_type=f32)
        a_s = lax.dot_general(hp, gas_ref[l:l + 1, :],
                              (((1,), (1,)), ((), ())),
                              preferred_element_type=f32)
        a_d = lax.dot_general(gad_ref[l:l + 1, :], hp,
                              (((1,), (1,)), ((), ())),
                              preferred_element_type=f32)
        e = a_s + a_d
        e = jnp.where(e > 0.0, e, 0.2 * e)
        m = jnp.max(jnp.where(valid_edge, e, _NEG), axis=0,
                    keepdims=True)
        m = jnp.where(m > -1e29, m, 0.0)
        ex = cnt * jnp.exp(jnp.minimum(e - m, 0.0))
        num = lax.dot_general(ex, hp, (((0,), (0,)), ((), ())),
                              preferred_element_type=f32)
        den = lax.dot_general(ex, ones_col, (((0,), (0,)), ((), ())),
                              preferred_element_type=f32)
        out = num / (den + 1e-16) + gb_ref[l:l + 1, :]
        h = jnp.maximum(out, 0.0) if l < nlayers - 1 else out

    x = x + h
    mu = jnp.mean(x, axis=1, keepdims=True)
    xc = x - mu
    var = jnp.mean(xc * xc, axis=1, keepdims=True)
    xn = xc * lax.rsqrt(var + 1e-5)
    logit = jnp.dot(xn, wsel_ref[...], preferred_element_type=f32) + cc
    out_ref[...] = vcol / (1.0 + jnp.exp(-logit))


def kernel(adj_matrix, timestep, arrival, departure, is_hard_to_match,
           total_timesteps, mask, W_emb1, b_emb1, W_emb2, b_emb2, gat_W,
           gat_att_src, gat_att_dst, gat_bias, W_sel, b_sel):
    n = adj_matrix.shape[0]
    hdim = W_emb2.shape[0]
    f32 = jnp.float32

    tsf = jnp.asarray(timestep, f32)
    ttf = jnp.asarray(total_timesteps, f32)
    cc = (tsf / ttf) * W_sel[hdim, 0] + b_sel[0]
    scal = jnp.stack([tsf, cc]).reshape(1, 2)

    vcol = (mask > 0).astype(f32).reshape(n, 1)
    vrow = vcol.reshape(1, n)

    out = pl.pallas_call(
        _model_body,
        out_shape=jax.ShapeDtypeStruct((n, 1), f32),
        compiler_params=pltpu.CompilerParams(
            vmem_limit_bytes=128 * 1024 * 1024),
    )(adj_matrix, scal, arrival.reshape(n, 1), departure.reshape(n, 1),
      is_hard_to_match.reshape(n, 1), vcol, vrow, W_emb1,
      b_emb1.reshape(1, hdim), W_emb2, b_emb2.reshape(1, hdim), gat_W,
      gat_att_src, gat_att_dst, gat_bias, W_sel[:hdim, :])
    return out


# speedup vs baseline: 10215.7677x; 10215.7677x over previous
import jax
import jax.numpy as jnp
from jax import lax
from jax.experimental import pallas as pl
from jax.experimental.pallas import tpu as pltpu

_NEG = -1e30


def _model_body(adj_ref, scal_ref, arr_ref, dep_ref, ihtm_ref, vcol_ref,
                vrow_ref, we1_ref, be1_ref, we2_ref, be2_ref, gw_ref,
                gas_ref, gad_ref, gb_ref, wsel_ref, out_ref):
    f32 = jnp.float32
    tsf = scal_ref[0, 0]
    cc = scal_ref[0, 1]

    arr = arr_ref[...]
    dep = dep_ref[...]
    ihtm = ihtm_ref[...]
    vcol = vcol_ref[...]
    vrow = vrow_ref[...]

    prog = (tsf - arr) / (dep - arr)
    x = prog * we1_ref[0:1, :] + ihtm * we1_ref[1:2, :] + be1_ref[...]
    x = jnp.dot(x, we2_ref[...], preferred_element_type=f32) + be2_ref[...]

    n = arr.shape[0]
    adjb = (adj_ref[...] > 0.0).astype(f32)
    ri = lax.broadcasted_iota(jnp.int32, (n, n), 0)
    ci = lax.broadcasted_iota(jnp.int32, (n, n), 1)
    diag = (ri == ci).astype(f32)
    cnt = adjb * (vcol * vrow) + diag * vrow
    valid_edge = cnt > 0.0

    ones_col = jnp.ones((n, 1), f32)

    h = x
    nlayers = gw_ref.shape[0]
    for l in range(nlayers):
        hp = jnp.dot(h, gw_ref[l], preferred_element_type=f32)
        a_s = lax.dot_general(hp, gas_ref[l:l + 1, :],
                              (((1,), (1,)), ((), ())),
                              preferred_element_type=f32)
        a_d = lax.dot_general(gad_ref[l:l + 1, :], hp,
                              (((1,), (1,)), ((), ())),
                              preferred_element_type=f32)
        e = a_s + a_d
        e = jnp.where(e > 0.0, e, 0.2 * e)
        m = jnp.max(jnp.where(valid_edge, e, _NEG), axis=0,
                    keepdims=True)
        m = jnp.where(m > -1e29, m, 0.0)
        ex = cnt * jnp.exp(jnp.minimum(e - m, 0.0))
        num = lax.dot_general(ex, hp, (((0,), (0,)), ((), ())),
                              preferred_element_type=f32)
        den = lax.dot_general(ex, ones_col, (((0,), (0,)), ((), ())),
                              preferred_element_type=f32)
        out = num / (den + 1e-16) + gb_ref[l:l + 1, :]
        h = jnp.maximum(out, 0.0) if l < nlayers - 1 else out

    x = x + h
    mu = jnp.mean(x, axis=1, keepdims=True)
    xc = x - mu
    var = jnp.mean(xc * xc, axis=1, keepdims=True)
    xn = xc * lax.rsqrt(var + 1e-5)
    logit = jnp.dot(xn, wsel_ref[...], preferred_element_type=f32) + cc
    out_ref[...] = vcol / (1.0 + jnp.exp(-logit))


def kernel(adj_matrix, timestep, arrival, departure, is_hard_to_match,
           total_timesteps, mask, W_emb1, b_emb1, W_emb2, b_emb2, gat_W,
           gat_att_src, gat_att_dst, gat_bias, W_sel, b_sel):
    n = adj_matrix.shape[0]
    hdim = W_emb2.shape[0]
    f32 = jnp.float32

    tsf = jnp.asarray(timestep, f32)
    ttf = jnp.asarray(total_timesteps, f32)
    cc = (tsf / ttf) * W_sel[hdim, 0] + b_sel[0]
    scal = jnp.stack([tsf, cc]).reshape(1, 2)

    vcol = (mask > 0).astype(f32).reshape(n, 1)
    vrow = vcol.reshape(1, n)

    out = pl.pallas_call(
        _model_body,
        out_shape=jax.ShapeDtypeStruct((n, 1), f32),
        compiler_params=pltpu.CompilerParams(
            vmem_limit_bytes=128 * 1024 * 1024),
    )(adj_matrix, scal, arrival.reshape(n, 1), departure.reshape(n, 1),
      is_hard_to_match.reshape(n, 1), vcol, vrow, W_emb1,
      b_emb1.reshape(1, hdim), W_emb2, b_emb2.reshape(1, hdim), gat_W,
      gat_att_src, gat_att_dst, gat_bias, W_sel[:hdim, :])
    return out
